# speedup vs baseline: 1.3214x; 1.3214x over previous
import jax
import jax.numpy as jnp
from jax.experimental import pallas as pl
from jax.experimental.pallas import tpu as pltpu

N = 8192
BLK1 = 512
BLK2 = 256


def _mlp_kernel(x_ref, w1_ref, b1_ref, w2_ref, b2_ref, w3_ref, b3_ref,
                h_ref, ht_ref):
    h = jnp.dot(x_ref[...], w1_ref[...], preferred_element_type=jnp.float32)
    h = jax.nn.relu(h + b1_ref[...])
    h = jnp.dot(h, w2_ref[...], preferred_element_type=jnp.float32)
    h = jax.nn.relu(h + b2_ref[...])
    h = jnp.dot(h, w3_ref[...], preferred_element_type=jnp.float32)
    h = jax.nn.relu(h + b3_ref[...])
    h_ref[...] = h
    ht_ref[...] = h.T


def _attn_kernel(h_ref, ht_ref, htblk_ref, wq_ref, wk_ref, bq_ref,
                 wv_ref, bvc_ref, gamma_ref, wa1_ref, ba1c_ref,
                 wa2_ref, ba2_ref, h2t_ref, s_ref):
    gt = jax.lax.dot_general(wk_ref[...], wq_ref[...],
                             (((1,), (1,)), ((), ())),
                             preferred_element_type=jnp.float32)
    ut = jax.lax.dot_general(wk_ref[...], bq_ref[...],
                             (((1,), (1,)), ((), ())),
                             preferred_element_type=jnp.float32)
    ft = jnp.dot(gt, htblk_ref[...], preferred_element_type=jnp.float32) + ut
    sc = jnp.dot(h_ref[...], ft, preferred_element_type=jnp.float32)
    m = jnp.max(sc, axis=0, keepdims=True)
    p = jnp.exp(sc - m)
    l = jnp.sum(p, axis=0, keepdims=True)
    ot = jnp.dot(ht_ref[...], p, preferred_element_type=jnp.float32) / l
    avt = jax.lax.dot_general(wv_ref[...], ot, (((0,), (0,)), ((), ())),
                              preferred_element_type=jnp.float32) + bvc_ref[...]
    h2t = gamma_ref[0, 0] * avt + htblk_ref[...]
    h2t_ref[...] = h2t
    tt = jnp.tanh(jax.lax.dot_general(wa1_ref[...], h2t, (((0,), (0,)), ((), ())),
                                      preferred_element_type=jnp.float32)
                  + ba1c_ref[...])
    s_ref[...] = jax.lax.dot_general(wa2_ref[...], tt, (((0,), (0,)), ((), ())),
                                     preferred_element_type=jnp.float32) + ba2_ref[...]


def _pool_kernel(s_ref, h2t_ref, wc_ref, bc_ref, y_ref, m_ref):
    s = s_ref[...]
    mx = jnp.max(s, axis=1, keepdims=True)
    e = jnp.exp(s - mx)
    z = jnp.sum(e, axis=1, keepdims=True)
    mnum = jax.lax.dot_general(h2t_ref[...], e, (((1,), (1,)), ((), ())),
                               preferred_element_type=jnp.float32)
    mcol = mnum / z
    m_ref[...] = mcol
    y = jax.lax.dot_general(mcol, wc_ref[...], (((0,), (0,)), ((), ())),
                            preferred_element_type=jnp.float32)
    y = jax.nn.sigmoid(y + bc_ref[...])
    y_ref[...] = jnp.clip(y, 1e-5, 1.0 - 1e-5)


def kernel(x, W1, b1, W2, b2, W3, b3, Wq, bq, Wk, bk, Wv, bv, gamma,
           Wa1, ba1, Wa2, ba2, Wc, bc):
    f32 = jnp.float32
    n_blk1 = N // BLK1
    n_blk2 = N // BLK2

    h, ht = pl.pallas_call(
        _mlp_kernel,
        grid=(n_blk1,),
        in_specs=[
            pl.BlockSpec((BLK1, 1024), lambda i: (i, 0)),
            pl.BlockSpec((1024, 256), lambda i: (0, 0)),
            pl.BlockSpec((1, 256), lambda i: (0, 0)),
            pl.BlockSpec((256, 128), lambda i: (0, 0)),
            pl.BlockSpec((1, 128), lambda i: (0, 0)),
            pl.BlockSpec((128, 64), lambda i: (0, 0)),
            pl.BlockSpec((1, 64), lambda i: (0, 0)),
        ],
        out_specs=[
            pl.BlockSpec((BLK1, 64), lambda i: (i, 0)),
            pl.BlockSpec((64, BLK1), lambda i: (0, i)),
        ],
        out_shape=[
            jax.ShapeDtypeStruct((N, 64), f32),
            jax.ShapeDtypeStruct((64, N), f32),
        ],
        compiler_params=pltpu.CompilerParams(
            dimension_semantics=("parallel",),
        ),
        name="sa_abmilp_mlp",
    )(x, W1, b1.reshape(1, 256), W2, b2.reshape(1, 128), W3, b3.reshape(1, 64))

    h2t, s = pl.pallas_call(
        _attn_kernel,
        grid=(n_blk2,),
        in_specs=[
            pl.BlockSpec((N, 64), lambda j: (0, 0)),
            pl.BlockSpec((64, N), lambda j: (0, 0)),
            pl.BlockSpec((64, BLK2), lambda j: (0, j)),
            pl.BlockSpec((64, 8), lambda j: (0, 0)),
            pl.BlockSpec((64, 8), lambda j: (0, 0)),
            pl.BlockSpec((1, 8), lambda j: (0, 0)),
            pl.BlockSpec((64, 64), lambda j: (0, 0)),
            pl.BlockSpec((64, 1), lambda j: (0, 0)),
            pl.BlockSpec((1, 1), lambda j: (0, 0)),
            pl.BlockSpec((64, 64), lambda j: (0, 0)),
            pl.BlockSpec((64, 1), lambda j: (0, 0)),
            pl.BlockSpec((64, 1), lambda j: (0, 0)),
            pl.BlockSpec((1, 1), lambda j: (0, 0)),
        ],
        out_specs=[
            pl.BlockSpec((64, BLK2), lambda j: (0, j)),
            pl.BlockSpec((1, BLK2), lambda j: (0, j)),
        ],
        out_shape=[
            jax.ShapeDtypeStruct((64, N), f32),
            jax.ShapeDtypeStruct((1, N), f32),
        ],
        compiler_params=pltpu.CompilerParams(
            dimension_semantics=("parallel",),
            vmem_limit_bytes=64 * 1024 * 1024,
        ),
        name="sa_abmilp_attn",
    )(h, ht, ht, Wq, Wk, bq.reshape(1, 8), Wv, bv.reshape(64, 1),
      gamma.reshape(1, 1), Wa1, ba1.reshape(64, 1), Wa2, ba2.reshape(1, 1))

    y, m = pl.pallas_call(
        _pool_kernel,
        out_shape=[
            jax.ShapeDtypeStruct((1, 1), f32),
            jax.ShapeDtypeStruct((64, 1), f32),
        ],
        name="sa_abmilp_pool",
    )(s, h2t, Wc, bc.reshape(1, 1))

    return (y[0, 0], m[:, 0])
